# baseline (speedup 1.0000x reference)
import jax
import jax.numpy as jnp
from jax.experimental import pallas as pl
from jax.experimental.pallas import tpu as pltpu

_S = 64.0
_MARGIN = 0.5
_C2 = _S * 1.4426950408889634


def _body_factory(B, C, BR, BC):
    NC = pl.cdiv(C, BC)
    NSL = BC // 128

    def body(x_ref, lab_ref, out_ref, m_ref, s_ref, pacc_ref):
        i = pl.program_id(0)
        j = pl.program_id(1)

        @pl.when(j == 0)
        def _init():
            m_ref[...] = jnp.full((BR, 1), -jnp.inf, jnp.float32)
            s_ref[...] = jnp.zeros((BR, 1), jnp.float32)
            pacc_ref[...] = jnp.zeros((BR, 128), jnp.float32)

        lab = lab_ref[...]
        lanes = jax.lax.broadcasted_iota(jnp.int32, (BR, 128), 1)

        def _block(masked):
            macc = jnp.full((BR, 128), -jnp.inf, jnp.float32)
            xs = []
            for k in range(NSL):
                xk = x_ref[:, k * 128:(k + 1) * 128]
                if masked:
                    gcol = j * BC + k * 128 + lanes
                    xk = jnp.where(gcol < C, xk, -jnp.inf)
                xs.append(xk)
                macc = jnp.maximum(macc, xk)
                rel = lab - (j * BC + k * 128)
                pacc_ref[...] += jnp.where(lanes == rel, xk, 0.0)
            m_old = m_ref[...]
            m_new = jnp.maximum(m_old, jnp.max(macc, axis=1, keepdims=True))
            mc = m_new * _C2
            sacc = jnp.zeros((BR, 128), jnp.float32)
            for xk in xs:
                sacc = sacc + jnp.exp2(xk * _C2 - mc)
            bs = jnp.sum(sacc, axis=1, keepdims=True)
            s_ref[...] = s_ref[...] * jnp.exp2(m_old * _C2 - mc) + bs
            m_ref[...] = m_new

        if C % BC != 0:
            @pl.when(j < NC - 1)
            def _fast():
                _block(False)

            @pl.when(j == NC - 1)
            def _slow():
                _block(True)
        else:
            _block(False)

        @pl.when(j == NC - 1)
        def _finish():
            m = m_ref[...]
            s = s_ref[...]
            a_x = jnp.sum(pacc_ref[...], axis=1, keepdims=True)
            q = jnp.exp(-_S * _MARGIN)
            ea = jnp.exp2(a_x * _C2 - m * _C2)
            s_adj = jnp.maximum(s - ea * (1.0 - q), ea * q)
            row_loss = _S * m + jnp.log(s_adj) - _S * (a_x - _MARGIN)
            tot = jnp.sum(row_loss) * (1.0 / B)

            @pl.when(i == 0)
            def _first():
                out_ref[...] = jnp.full((1, 1), tot, jnp.float32)

            @pl.when(i != 0)
            def _rest():
                out_ref[...] = out_ref[...] + tot

    return body, NC


def _grid_call(cosine, lab2d, BR, BC):
    B, C = cosine.shape
    body, NC = _body_factory(B, C, BR, BC)
    out = pl.pallas_call(
        body,
        grid=(B // BR, NC),
        in_specs=[
            pl.BlockSpec((BR, BC), lambda i, j: (i, j)),
            pl.BlockSpec((BR, 1), lambda i, j: (i, 0)),
        ],
        out_specs=pl.BlockSpec((1, 1), lambda i, j: (0, 0)),
        out_shape=jax.ShapeDtypeStruct((1, 1), jnp.float32),
        scratch_shapes=[
            pltpu.VMEM((BR, 1), jnp.float32),
            pltpu.VMEM((BR, 1), jnp.float32),
            pltpu.VMEM((BR, 128), jnp.float32),
        ],
        compiler_params=pltpu.CompilerParams(
            dimension_semantics=("arbitrary", "arbitrary")
        ),
    )(cosine, lab2d)
    return out[0, 0]


@jax.jit
def kernel(cosine, label):
    B, _ = cosine.shape
    lab2d = label.astype(jnp.int32).reshape(B, 1)
    return _grid_call(cosine, lab2d, 512, 4096)


# speedup vs baseline: 1.1375x; 1.1375x over previous
import jax
import jax.numpy as jnp
from jax.experimental import pallas as pl
from jax.experimental.pallas import tpu as pltpu

_S = 64.0
_MARGIN = 0.5
_C2 = _S * 1.4426950408889634


def _body_factory(B, C, BR, BC):
    NC = pl.cdiv(C, BC)
    NSL = BC // 128

    def body(x_ref, ax_ref, out_ref, m_ref, s_ref):
        i = pl.program_id(0)
        j = pl.program_id(1)

        @pl.when(j == 0)
        def _init():
            m_ref[...] = jnp.full((BR, 1), -jnp.inf, jnp.float32)
            s_ref[...] = jnp.zeros((BR, 1), jnp.float32)

        def _slice(k, masked):
            xk = x_ref[:, k * 128:(k + 1) * 128]
            if masked:
                lanes = jax.lax.broadcasted_iota(jnp.int32, (BR, 128), 1)
                gcol = j * BC + k * 128 + lanes
                xk = jnp.where(gcol < C, xk, -jnp.inf)
            return xk

        def _block(masked):
            macc = jnp.full((BR, 128), -jnp.inf, jnp.float32)
            for k in range(NSL):
                macc = jnp.maximum(macc, _slice(k, masked))
            m_old = m_ref[...]
            m_new = jnp.maximum(m_old, jnp.max(macc, axis=1, keepdims=True))
            mc = m_new * _C2
            sacc = jnp.zeros((BR, 128), jnp.float32)
            for k in range(NSL):
                sacc = sacc + jnp.exp2(_slice(k, masked) * _C2 - mc)
            bs = jnp.sum(sacc, axis=1, keepdims=True)
            s_ref[...] = s_ref[...] * jnp.exp2(m_old * _C2 - mc) + bs
            m_ref[...] = m_new

        if C % BC != 0:
            @pl.when(j < NC - 1)
            def _fast():
                _block(False)

            @pl.when(j == NC - 1)
            def _slow():
                _block(True)
        else:
            _block(False)

        @pl.when(j == NC - 1)
        def _finish():
            m = m_ref[...]
            s = s_ref[...]
            a_x = ax_ref[...]
            q = jnp.exp(-_S * _MARGIN)
            ea = jnp.exp2(a_x * _C2 - m * _C2)
            s_adj = jnp.maximum(s - ea * (1.0 - q), ea * q)
            row_loss = _S * m + jnp.log(s_adj) - _S * (a_x - _MARGIN)
            tot = jnp.sum(row_loss) * (1.0 / B)

            @pl.when(i == 0)
            def _first():
                out_ref[...] = jnp.full((1, 1), tot, jnp.float32)

            @pl.when(i != 0)
            def _rest():
                out_ref[...] = out_ref[...] + tot

    return body, NC


def _grid_call(cosine, ax2d, BR, BC):
    B, C = cosine.shape
    body, NC = _body_factory(B, C, BR, BC)
    out = pl.pallas_call(
        body,
        grid=(B // BR, NC),
        in_specs=[
            pl.BlockSpec((BR, BC), lambda i, j: (i, j)),
            pl.BlockSpec((BR, 1), lambda i, j: (i, 0)),
        ],
        out_specs=pl.BlockSpec((1, 1), lambda i, j: (0, 0)),
        out_shape=jax.ShapeDtypeStruct((1, 1), jnp.float32),
        scratch_shapes=[
            pltpu.VMEM((BR, 1), jnp.float32),
            pltpu.VMEM((BR, 1), jnp.float32),
        ],
        compiler_params=pltpu.CompilerParams(
            dimension_semantics=("arbitrary", "arbitrary")
        ),
    )(cosine, ax2d)
    return out[0, 0]


@jax.jit
def kernel(cosine, label):
    B, _ = cosine.shape
    lab = label.astype(jnp.int32)
    ax2d = jnp.take_along_axis(cosine, lab[:, None], axis=1)
    return _grid_call(cosine, ax2d, 512, 4096)
